# baseline (speedup 1.0000x reference)
import functools

import jax
import jax.numpy as jnp
from jax.experimental import pallas as pl

_TL = 1024


def _pe_add_kernel(x_ref, pe_ref, o_ref, *, blocks_per_l: int):
    i = pl.program_id(0)
    off = (i % blocks_per_l) * _TL
    o_ref[...] = x_ref[...] + pe_ref[pl.ds(off, _TL), :]


def kernel(x, pe_weight):
    b, l, d = x.shape
    xf = x.reshape(b * l, d)
    out = pl.pallas_call(
        functools.partial(_pe_add_kernel, blocks_per_l=l // _TL),
        grid=(b * l // _TL,),
        in_specs=[
            pl.BlockSpec((_TL, d), lambda i: (i, 0)),
            pl.BlockSpec((l, d), lambda i: (0, 0)),
        ],
        out_specs=pl.BlockSpec((_TL, d), lambda i: (i, 0)),
        out_shape=jax.ShapeDtypeStruct((b * l, d), x.dtype),
    )(xf, pe_weight)
    return out.reshape(b, l, d)


# speedup vs baseline: 1.0328x; 1.0328x over previous
import functools

import jax
import jax.numpy as jnp
from jax.experimental import pallas as pl

_TL = 2048


def _pe_add_kernel(x_ref, pe_ref, o_ref, *, blocks_per_l: int):
    i = pl.program_id(0)
    off = (i % blocks_per_l) * _TL
    o_ref[...] = x_ref[...] + pe_ref[pl.ds(off, _TL), :]


def kernel(x, pe_weight):
    b, l, d = x.shape
    xf = x.reshape(b * l, d)
    out = pl.pallas_call(
        functools.partial(_pe_add_kernel, blocks_per_l=l // _TL),
        grid=(b * l // _TL,),
        in_specs=[
            pl.BlockSpec((_TL, d), lambda i: (i, 0)),
            pl.BlockSpec((l, d), lambda i: (0, 0)),
        ],
        out_specs=pl.BlockSpec((_TL, d), lambda i: (i, 0)),
        out_shape=jax.ShapeDtypeStruct((b * l, d), x.dtype),
    )(xf, pe_weight)
    return out.reshape(b, l, d)
